# Initial kernel scaffold; baseline (speedup 1.0000x reference)
import jax
import jax.numpy as jnp
from jax.experimental import pallas as pl


def kernel(pts, feats, W1, b1, W2, b2):
    raise NotImplementedError("write your pallas kernel here")


# speedup vs baseline: 14.3854x; 14.3854x over previous
import functools

import jax
import jax.numpy as jnp
from jax import lax
from jax.experimental import pallas as pl
from jax.experimental.pallas import tpu as pltpu
from jax.experimental.pallas import tpu_sc as plsc


def _pre_body(pts_ref, feats_ref, w1a_ref, w1b_ref, b1_ref, gp_ref, aq_ref):
    p = pts_ref[0]
    f = feats_ref[0]
    a = jnp.dot(p, w1a_ref[...], preferred_element_type=jnp.float32)
    g = jnp.dot(f, w1b_ref[...], preferred_element_type=jnp.float32)
    aq_ref[0] = a
    gp_ref[0] = g + a + b1_ref[...]


def _precompute(pts, feats, w1a, w1b, b1r):
    B, P, _ = pts.shape
    C = w1b.shape[1]
    return pl.pallas_call(
        _pre_body,
        grid=(B,),
        in_specs=[
            pl.BlockSpec((1, P, 3), lambda b: (b, 0, 0)),
            pl.BlockSpec((1, P, C), lambda b: (b, 0, 0)),
            pl.BlockSpec((3, C), lambda b: (0, 0)),
            pl.BlockSpec((C, C), lambda b: (0, 0)),
            pl.BlockSpec((1, C), lambda b: (0, 0)),
        ],
        out_specs=[
            pl.BlockSpec((1, P, C), lambda b: (b, 0, 0)),
            pl.BlockSpec((1, P, C), lambda b: (b, 0, 0)),
        ],
        out_shape=[
            jax.ShapeDtypeStruct((B, P, C), jnp.float32),
            jax.ShapeDtypeStruct((B, P, C), jnp.float32),
        ],
    )(pts, feats, w1a, w1b, b1r)


def _topk_body(K, P, ptsq_ref, ptsT_ref, idx_ref):
    b = pl.program_id(0)
    q = ptsq_ref[0]
    pT = ptsT_ref[0]
    BQ = q.shape[0]
    d = None
    for c in range(3):
        t = q[:, c:c + 1] - pT[c:c + 1, :]
        t = t * t
        d = t if d is None else d + t
    iota = lax.broadcasted_iota(jnp.int32, (BQ, P), 1)
    inf = jnp.float32(float("inf"))
    base = b * P
    for k in range(K):
        m = jnp.min(d, axis=1, keepdims=True)
        im = jnp.min(jnp.where(d == m, iota, P), axis=1, keepdims=True)
        idx_ref[0, :, k:k + 1] = im + base
        d = jnp.where(iota == im, inf, d)


def _topk(pts, ptsT, K, BQ):
    B, P, _ = pts.shape
    return pl.pallas_call(
        functools.partial(_topk_body, K, P),
        grid=(B, P // BQ),
        in_specs=[
            pl.BlockSpec((1, BQ, 3), lambda b, i: (b, i, 0)),
            pl.BlockSpec((1, 3, P), lambda b, i: (b, 0, 0)),
        ],
        out_specs=pl.BlockSpec((1, BQ, K), lambda b, i: (b, i, 0)),
        out_shape=jax.ShapeDtypeStruct((B, P, K), jnp.int32),
    )(pts, ptsT)


def _sc_gather(table, idxg, CH=128):
    N = idxg.shape[0]
    C = table.shape[1]
    info = plsc.get_sparse_core_info()
    NC, NS = info.num_cores, info.num_subcores
    NW = NC * NS
    n_per_w = N // NW
    nch = n_per_w // CH
    mesh = plsc.VectorSubcoreMesh(core_axis_name="c", subcore_axis_name="s")

    @functools.partial(
        pl.kernel,
        out_type=jax.ShapeDtypeStruct((N, C), jnp.float32),
        mesh=mesh,
        scratch_types=[
            pltpu.VMEM((CH,), jnp.int32),
            pltpu.VMEM((CH, C), jnp.float32),
            pltpu.SemaphoreType.DMA,
        ],
    )
    def k(table_hbm, idx_hbm, out_hbm, idx_v, rows_v, sem):
        wid = lax.axis_index("s") * NC + lax.axis_index("c")
        base = wid * n_per_w

        def body(i, _):
            off = base + i * CH
            pltpu.sync_copy(idx_hbm.at[pl.ds(off, CH)], idx_v)
            pltpu.async_copy(table_hbm.at[idx_v], rows_v, sem).wait()
            pltpu.sync_copy(rows_v, out_hbm.at[pl.ds(off, CH)])
            return 0

        lax.fori_loop(0, nch, body, 0)

    return k(table, idxg)


def _mlp_body(K, C, nf_ref, aq_ref, w2_ref, b2_ref, out_ref):
    BQ = aq_ref.shape[0]
    nf = nf_ref[...].reshape(BQ, K, C)
    aq = aq_ref[...]
    h = jnp.maximum(nf - aq[:, None, :], 0.0)
    u = jnp.dot(h.reshape(BQ * K, C), w2_ref[...],
                preferred_element_type=jnp.float32) + b2_ref[...]
    out_ref[...] = jnp.max(u.reshape(BQ, K, C), axis=1)


def _mlp(NF, Aq2, W2, b2r, K, BQ):
    NT, C = Aq2.shape
    return pl.pallas_call(
        functools.partial(_mlp_body, K, C),
        grid=(NT // BQ,),
        in_specs=[
            pl.BlockSpec((BQ * K, C), lambda i: (i, 0)),
            pl.BlockSpec((BQ, C), lambda i: (i, 0)),
            pl.BlockSpec((C, C), lambda i: (0, 0)),
            pl.BlockSpec((1, C), lambda i: (0, 0)),
        ],
        out_specs=pl.BlockSpec((BQ, C), lambda i: (i, 0)),
        out_shape=jax.ShapeDtypeStruct((NT, C), jnp.float32),
    )(NF, Aq2, W2, b2r)


def kernel(pts, feats, W1, b1, W2, b2):
    B, P, _ = pts.shape
    C = feats.shape[-1]
    K = 16
    w1a = W1[:3]
    w1b = W1[3:]
    ptsT = jnp.transpose(pts, (0, 2, 1))
    Gp, Aq = _precompute(pts, feats, w1a, w1b, b1.reshape(1, C))
    idxg = _topk(pts, ptsT, K, BQ=256)
    NF = _sc_gather(Gp.reshape(B * P, C), idxg.reshape(B * P * K))
    out = _mlp(NF, Aq.reshape(B * P, C), W2, b2.reshape(1, C), K, BQ=128)
    return out.reshape(B, P, C)


# speedup vs baseline: 15.6175x; 1.0857x over previous
import functools

import jax
import jax.numpy as jnp
from jax import lax
from jax.experimental import pallas as pl
from jax.experimental.pallas import tpu as pltpu
from jax.experimental.pallas import tpu_sc as plsc


def _pre_body(pts_ref, feats_ref, w1a_ref, w1b_ref, b1_ref, gp_ref, aq_ref):
    p = pts_ref[0]
    f = feats_ref[0]
    a = jnp.dot(p, w1a_ref[...], preferred_element_type=jnp.float32)
    g = jnp.dot(f, w1b_ref[...], preferred_element_type=jnp.float32)
    aq_ref[0] = a
    gp_ref[0] = g + a + b1_ref[...]


def _precompute(pts, feats, w1a, w1b, b1r):
    B, P, _ = pts.shape
    C = w1b.shape[1]
    return pl.pallas_call(
        _pre_body,
        grid=(B,),
        in_specs=[
            pl.BlockSpec((1, P, 3), lambda b: (b, 0, 0)),
            pl.BlockSpec((1, P, C), lambda b: (b, 0, 0)),
            pl.BlockSpec((3, C), lambda b: (0, 0)),
            pl.BlockSpec((C, C), lambda b: (0, 0)),
            pl.BlockSpec((1, C), lambda b: (0, 0)),
        ],
        out_specs=[
            pl.BlockSpec((1, P, C), lambda b: (b, 0, 0)),
            pl.BlockSpec((1, P, C), lambda b: (b, 0, 0)),
        ],
        out_shape=[
            jax.ShapeDtypeStruct((B, P, C), jnp.float32),
            jax.ShapeDtypeStruct((B, P, C), jnp.float32),
        ],
    )(pts, feats, w1a, w1b, b1r)


def _topk_body(K, P, base, ptsq_ref, ptsT_ref, idx_ref):
    q = ptsq_ref[0]
    pT = ptsT_ref[0]
    BQ = q.shape[0]
    d = None
    for c in range(3):
        t = q[:, c:c + 1] - pT[c:c + 1, :]
        t = t * t
        d = t if d is None else d + t
    iota = lax.broadcasted_iota(jnp.int32, (BQ, P), 1)
    inf = jnp.float32(float("inf"))
    for k in range(K):
        m = jnp.min(d, axis=1, keepdims=True)
        im = jnp.min(jnp.where(d == m, iota, P), axis=1, keepdims=True)
        idx_ref[0, :, k:k + 1] = im + base
        d = jnp.where(iota == im, inf, d)


def _topk(pts, ptsT, K, BQ, base=0):
    B, P, _ = pts.shape
    return pl.pallas_call(
        functools.partial(_topk_body, K, P, base),
        grid=(B, P // BQ),
        in_specs=[
            pl.BlockSpec((1, BQ, 3), lambda b, i: (b, i, 0)),
            pl.BlockSpec((1, 3, P), lambda b, i: (b, 0, 0)),
        ],
        out_specs=pl.BlockSpec((1, BQ, K), lambda b, i: (b, i, 0)),
        out_shape=jax.ShapeDtypeStruct((B, P, K), jnp.int32),
    )(pts, ptsT)


def _sc_gather(table, idxg, CH=128):
    N = idxg.shape[0]
    C = table.shape[1]
    info = plsc.get_sparse_core_info()
    NC, NS = info.num_cores, info.num_subcores
    NW = NC * NS
    n_per_w = N // NW
    nch = n_per_w // CH
    mesh = plsc.VectorSubcoreMesh(core_axis_name="c", subcore_axis_name="s")

    @functools.partial(
        pl.kernel,
        out_type=jax.ShapeDtypeStruct((N, C), jnp.float32),
        mesh=mesh,
        scratch_types=[
            pltpu.VMEM((CH,), jnp.int32),
            pltpu.VMEM((CH, C), jnp.float32),
            pltpu.SemaphoreType.DMA,
        ],
    )
    def k(table_hbm, idx_hbm, out_hbm, idx_v, rows_v, sem):
        wid = lax.axis_index("s") * NC + lax.axis_index("c")
        base = wid * n_per_w

        def body(i, _):
            off = base + i * CH
            pltpu.sync_copy(idx_hbm.at[pl.ds(off, CH)], idx_v)
            pltpu.async_copy(table_hbm.at[idx_v], rows_v, sem).wait()
            pltpu.sync_copy(rows_v, out_hbm.at[pl.ds(off, CH)])
            return 0

        lax.fori_loop(0, nch, body, 0)

    return k(table, idxg)


def _mlp_body(K, C, nf_ref, aq_ref, w2_ref, b2_ref, out_ref):
    BQ = aq_ref.shape[0]
    nf = nf_ref[...].reshape(BQ, K, C)
    aq = aq_ref[...]
    h = jnp.maximum(nf - aq[:, None, :], 0.0)
    u = jnp.dot(h.reshape(BQ * K, C), w2_ref[...],
                preferred_element_type=jnp.float32) + b2_ref[...]
    out_ref[...] = jnp.max(u.reshape(BQ, K, C), axis=1)


def _mlp(NF, Aq2, W2, b2r, K, BQ):
    NT, C = Aq2.shape
    return pl.pallas_call(
        functools.partial(_mlp_body, K, C),
        grid=(NT // BQ,),
        in_specs=[
            pl.BlockSpec((BQ * K, C), lambda i: (i, 0)),
            pl.BlockSpec((BQ, C), lambda i: (i, 0)),
            pl.BlockSpec((C, C), lambda i: (0, 0)),
            pl.BlockSpec((1, C), lambda i: (0, 0)),
        ],
        out_specs=pl.BlockSpec((BQ, C), lambda i: (i, 0)),
        out_shape=jax.ShapeDtypeStruct((NT, C), jnp.float32),
    )(NF, Aq2, W2, b2r)


def kernel(pts, feats, W1, b1, W2, b2):
    B, P, _ = pts.shape
    C = feats.shape[-1]
    K = 16
    w1a = W1[:3]
    w1b = W1[3:]
    ptsT = jnp.transpose(pts, (0, 2, 1))
    Gp, Aq = _precompute(pts, feats, w1a, w1b, b1.reshape(1, C))
    Gp2 = Gp.reshape(B * P, C)
    Aq3 = Aq.reshape(B, P, C)
    outs = []
    for b in range(B):
        idxb = _topk(pts[b:b + 1], ptsT[b:b + 1], K, BQ=256, base=b * P)
        NFb = _sc_gather(Gp2, idxb.reshape(P * K))
        outs.append(_mlp(NFb, Aq3[b], W2, b2.reshape(1, C), K, BQ=128))
    return jnp.stack(outs, axis=0)


# speedup vs baseline: 17.4038x; 1.1144x over previous
import functools

import jax
import jax.numpy as jnp
from jax import lax
from jax.experimental import pallas as pl
from jax.experimental.pallas import tpu as pltpu
from jax.experimental.pallas import tpu_sc as plsc


def _pre_body(pts_ref, feats_ref, w1a_ref, w1b_ref, b1_ref, gp_ref, aq_ref):
    p = pts_ref[0]
    f = feats_ref[0]
    a = jnp.dot(p, w1a_ref[...], preferred_element_type=jnp.float32)
    g = jnp.dot(f, w1b_ref[...], preferred_element_type=jnp.float32)
    aq_ref[0] = a
    gp_ref[0] = g + a + b1_ref[...]


def _precompute(pts, feats, w1a, w1b, b1r):
    B, P, _ = pts.shape
    C = w1b.shape[1]
    return pl.pallas_call(
        _pre_body,
        grid=(B,),
        in_specs=[
            pl.BlockSpec((1, P, 3), lambda b: (b, 0, 0)),
            pl.BlockSpec((1, P, C), lambda b: (b, 0, 0)),
            pl.BlockSpec((3, C), lambda b: (0, 0)),
            pl.BlockSpec((C, C), lambda b: (0, 0)),
            pl.BlockSpec((1, C), lambda b: (0, 0)),
        ],
        out_specs=[
            pl.BlockSpec((1, P, C), lambda b: (b, 0, 0)),
            pl.BlockSpec((1, P, C), lambda b: (b, 0, 0)),
        ],
        out_shape=[
            jax.ShapeDtypeStruct((B, P, C), jnp.float32),
            jax.ShapeDtypeStruct((B, P, C), jnp.float32),
        ],
    )(pts, feats, w1a, w1b, b1r)


def _topk_body(K, P, base, ptsq_ref, ptsT_ref, idx_ref):
    q = ptsq_ref[0]
    pT = ptsT_ref[0]
    BQ = q.shape[0]
    d = None
    for c in range(3):
        t = q[:, c:c + 1] - pT[c:c + 1, :]
        t = t * t
        d = t if d is None else d + t
    r_i = lax.broadcasted_iota(jnp.int32, (P, 8), 0)
    r_j = lax.broadcasted_iota(jnp.int32, (P, 8), 1)
    iomat = jnp.where(r_j == 0, r_i, 0).astype(jnp.float32)
    inf = jnp.float32(float("inf"))
    for k in range(K):
        m = jnp.min(d, axis=1, keepdims=True)
        eq = d == m
        imf = jnp.dot(eq.astype(jnp.float32), iomat,
                      preferred_element_type=jnp.float32)
        idx_ref[0, :, k:k + 1] = imf[:, :1].astype(jnp.int32) + base
        if k < K - 1:
            d = jnp.where(eq, inf, d)


def _topk(pts, ptsT, K, BQ, base=0):
    B, P, _ = pts.shape
    return pl.pallas_call(
        functools.partial(_topk_body, K, P, base),
        grid=(B, P // BQ),
        in_specs=[
            pl.BlockSpec((1, BQ, 3), lambda b, i: (b, i, 0)),
            pl.BlockSpec((1, 3, P), lambda b, i: (b, 0, 0)),
        ],
        out_specs=pl.BlockSpec((1, BQ, K), lambda b, i: (b, i, 0)),
        out_shape=jax.ShapeDtypeStruct((B, P, K), jnp.int32),
    )(pts, ptsT)


def _sc_gather(table, idxg, CH=128):
    N = idxg.shape[0]
    C = table.shape[1]
    info = plsc.get_sparse_core_info()
    NC, NS = info.num_cores, info.num_subcores
    NW = NC * NS
    n_per_w = N // NW
    nch = n_per_w // CH
    mesh = plsc.VectorSubcoreMesh(core_axis_name="c", subcore_axis_name="s")

    @functools.partial(
        pl.kernel,
        out_type=jax.ShapeDtypeStruct((N, C), jnp.float32),
        mesh=mesh,
        scratch_types=[
            pltpu.VMEM((CH,), jnp.int32),
            pltpu.VMEM((CH, C), jnp.float32),
            pltpu.SemaphoreType.DMA,
        ],
    )
    def k(table_hbm, idx_hbm, out_hbm, idx_v, rows_v, sem):
        wid = lax.axis_index("s") * NC + lax.axis_index("c")
        base = wid * n_per_w

        nrow = table_hbm.shape[0]

        def body(i, _):
            off = base + i * CH
            pltpu.sync_copy(idx_hbm.at[pl.ds(off, CH)], idx_v)
            for j in range(CH // 16):
                s = pl.ds(j * 16, 16)
                iv = idx_v[s]
                idx_v[s] = jnp.minimum(jnp.maximum(iv, 0), nrow - 1)
            pltpu.async_copy(table_hbm.at[idx_v], rows_v, sem).wait()
            pltpu.sync_copy(rows_v, out_hbm.at[pl.ds(off, CH)])
            return 0

        lax.fori_loop(0, nch, body, 0)

    return k(table, idxg)


def _mlp_body(K, C, nf_ref, aq_ref, w2_ref, b2_ref, out_ref):
    BQ = aq_ref.shape[0]
    nf = nf_ref[...].reshape(BQ, K, C)
    aq = aq_ref[...]
    h = jnp.maximum(nf - aq[:, None, :], 0.0)
    u = jnp.dot(h.reshape(BQ * K, C), w2_ref[...],
                preferred_element_type=jnp.float32) + b2_ref[...]
    out_ref[...] = jnp.max(u.reshape(BQ, K, C), axis=1)


def _mlp(NF, Aq2, W2, b2r, K, BQ):
    NT, C = Aq2.shape
    return pl.pallas_call(
        functools.partial(_mlp_body, K, C),
        grid=(NT // BQ,),
        in_specs=[
            pl.BlockSpec((BQ * K, C), lambda i: (i, 0)),
            pl.BlockSpec((BQ, C), lambda i: (i, 0)),
            pl.BlockSpec((C, C), lambda i: (0, 0)),
            pl.BlockSpec((1, C), lambda i: (0, 0)),
        ],
        out_specs=pl.BlockSpec((BQ, C), lambda i: (i, 0)),
        out_shape=jax.ShapeDtypeStruct((NT, C), jnp.float32),
    )(NF, Aq2, W2, b2r)


def kernel(pts, feats, W1, b1, W2, b2):
    B, P, _ = pts.shape
    C = feats.shape[-1]
    K = 16
    w1a = W1[:3]
    w1b = W1[3:]
    ptsT = jnp.transpose(pts, (0, 2, 1))
    Gp, Aq = _precompute(pts, feats, w1a, w1b, b1.reshape(1, C))
    Gp2 = Gp.reshape(B * P, C)
    Aq3 = Aq.reshape(B, P, C)
    outs = []
    for b in range(B):
        idxb = _topk(pts[b:b + 1], ptsT[b:b + 1], K, BQ=256, base=b * P)
        NFb = _sc_gather(Gp2, idxb.reshape(P * K))
        outs.append(_mlp(NFb, Aq3[b], W2, b2.reshape(1, C), K, BQ=128))
    return jnp.stack(outs, axis=0)


# speedup vs baseline: 17.8851x; 1.0277x over previous
import functools

import jax
import jax.numpy as jnp
from jax import lax
from jax.experimental import pallas as pl
from jax.experimental.pallas import tpu as pltpu
from jax.experimental.pallas import tpu_sc as plsc


def _pre_body(pts_ref, feats_ref, w1a_ref, w1b_ref, b1_ref, gp_ref, aq_ref):
    p = pts_ref[0]
    f = feats_ref[0]
    a = jnp.dot(p, w1a_ref[...], preferred_element_type=jnp.float32)
    g = jnp.dot(f, w1b_ref[...], preferred_element_type=jnp.float32)
    aq_ref[0] = a
    gp_ref[0] = g + a + b1_ref[...]


def _precompute(pts, feats, w1a, w1b, b1r):
    B, P, _ = pts.shape
    C = w1b.shape[1]
    return pl.pallas_call(
        _pre_body,
        grid=(B,),
        in_specs=[
            pl.BlockSpec((1, P, 3), lambda b: (b, 0, 0)),
            pl.BlockSpec((1, P, C), lambda b: (b, 0, 0)),
            pl.BlockSpec((3, C), lambda b: (0, 0)),
            pl.BlockSpec((C, C), lambda b: (0, 0)),
            pl.BlockSpec((1, C), lambda b: (0, 0)),
        ],
        out_specs=[
            pl.BlockSpec((1, P, C), lambda b: (b, 0, 0)),
            pl.BlockSpec((1, P, C), lambda b: (b, 0, 0)),
        ],
        out_shape=[
            jax.ShapeDtypeStruct((B, P, C), jnp.float32),
            jax.ShapeDtypeStruct((B, P, C), jnp.float32),
        ],
    )(pts, feats, w1a, w1b, b1r)


def _topk_body(K, P, base, ptsq_ref, ptsT_ref, idx_ref):
    q = ptsq_ref[0]
    pT = ptsT_ref[0]
    BQ = q.shape[0]
    d = None
    for c in range(3):
        t = q[:, c:c + 1] - pT[c:c + 1, :]
        t = t * t
        d = t if d is None else d + t
    r_i = lax.broadcasted_iota(jnp.int32, (P, 8), 0)
    r_j = lax.broadcasted_iota(jnp.int32, (P, 8), 1)
    iomat = jnp.where(r_j == 0, r_i >> 6,
                      jnp.where(r_j == 1, r_i & 63, 0)).astype(jnp.float32)
    inf = jnp.float32(float("inf"))
    for k in range(K):
        m = jnp.min(d, axis=1, keepdims=True)
        eq = d == m
        imf = jnp.dot(eq.astype(jnp.float32), iomat,
                      preferred_element_type=jnp.float32)
        im = imf[:, :1].astype(jnp.int32) * 64 + imf[:, 1:2].astype(jnp.int32)
        idx_ref[0, :, k:k + 1] = im + base
        if k < K - 1:
            d = jnp.where(eq, inf, d)


def _topk(pts, ptsT, K, BQ, base=0):
    B, P, _ = pts.shape
    return pl.pallas_call(
        functools.partial(_topk_body, K, P, base),
        grid=(B, P // BQ),
        in_specs=[
            pl.BlockSpec((1, BQ, 3), lambda b, i: (b, i, 0)),
            pl.BlockSpec((1, 3, P), lambda b, i: (b, 0, 0)),
        ],
        out_specs=pl.BlockSpec((1, BQ, K), lambda b, i: (b, i, 0)),
        out_shape=jax.ShapeDtypeStruct((B, P, K), jnp.int32),
    )(pts, ptsT)


def _sc_gather(table, idxg, CH=128):
    N = idxg.shape[0]
    C = table.shape[1]
    info = plsc.get_sparse_core_info()
    NC, NS = info.num_cores, info.num_subcores
    NW = NC * NS
    n_per_w = N // NW
    nch = n_per_w // CH
    mesh = plsc.VectorSubcoreMesh(core_axis_name="c", subcore_axis_name="s")

    @functools.partial(
        pl.kernel,
        out_type=jax.ShapeDtypeStruct((N, C), jnp.float32),
        mesh=mesh,
        scratch_types=[
            pltpu.VMEM((CH,), jnp.int32),
            pltpu.VMEM((CH, C), jnp.float32),
            pltpu.SemaphoreType.DMA,
        ],
    )
    def k(table_hbm, idx_hbm, out_hbm, idx_v, rows_v, sem):
        wid = lax.axis_index("s") * NC + lax.axis_index("c")
        base = wid * n_per_w

        nrow = table_hbm.shape[0]

        def body(i, _):
            off = base + i * CH
            pltpu.sync_copy(idx_hbm.at[pl.ds(off, CH)], idx_v)
            for j in range(CH // 16):
                s = pl.ds(j * 16, 16)
                iv = idx_v[s]
                idx_v[s] = jnp.minimum(jnp.maximum(iv, 0), nrow - 1)
            pltpu.async_copy(table_hbm.at[idx_v], rows_v, sem).wait()
            pltpu.sync_copy(rows_v, out_hbm.at[pl.ds(off, CH)])
            return 0

        lax.fori_loop(0, nch, body, 0)

    return k(table, idxg)


def _mlp_body(K, C, nf_ref, aq_ref, w2_ref, b2_ref, out_ref):
    BQ = aq_ref.shape[0]
    nf = nf_ref[...].reshape(BQ, K, C)
    aq = aq_ref[...]
    h = jnp.maximum(nf - aq[:, None, :], 0.0)
    u = jnp.dot(h.reshape(BQ * K, C), w2_ref[...],
                preferred_element_type=jnp.float32) + b2_ref[...]
    out_ref[...] = jnp.max(u.reshape(BQ, K, C), axis=1)


def _mlp(NF, Aq2, W2, b2r, K, BQ):
    NT, C = Aq2.shape
    return pl.pallas_call(
        functools.partial(_mlp_body, K, C),
        grid=(NT // BQ,),
        in_specs=[
            pl.BlockSpec((BQ * K, C), lambda i: (i, 0)),
            pl.BlockSpec((BQ, C), lambda i: (i, 0)),
            pl.BlockSpec((C, C), lambda i: (0, 0)),
            pl.BlockSpec((1, C), lambda i: (0, 0)),
        ],
        out_specs=pl.BlockSpec((BQ, C), lambda i: (i, 0)),
        out_shape=jax.ShapeDtypeStruct((NT, C), jnp.float32),
    )(NF, Aq2, W2, b2r)


def kernel(pts, feats, W1, b1, W2, b2):
    B, P, _ = pts.shape
    C = feats.shape[-1]
    K = 16
    w1a = W1[:3]
    w1b = W1[3:]
    ptsT = jnp.transpose(pts, (0, 2, 1))
    Gp, Aq = _precompute(pts, feats, w1a, w1b, b1.reshape(1, C))
    Gp2 = Gp.reshape(B * P, C)
    Aq3 = Aq.reshape(B, P, C)
    outs = []
    for b in range(B):
        idxb = _topk(pts[b:b + 1], ptsT[b:b + 1], K, BQ=256, base=b * P)
        NFb = _sc_gather(Gp2, idxb.reshape(P * K))
        outs.append(_mlp(NFb, Aq3[b], W2, b2.reshape(1, C), K, BQ=128))
    return jnp.stack(outs, axis=0)


# speedup vs baseline: 19.7321x; 1.1033x over previous
import functools

import jax
import jax.numpy as jnp
from jax import lax
from jax.experimental import pallas as pl
from jax.experimental.pallas import tpu as pltpu
from jax.experimental.pallas import tpu_sc as plsc


def _pre_body(pts_ref, feats_ref, w1a_ref, w1b_ref, b1_ref, gp_ref, aq_ref):
    p = pts_ref[0]
    f = feats_ref[0]
    a = jnp.dot(p, w1a_ref[...], preferred_element_type=jnp.float32)
    g = jnp.dot(f, w1b_ref[...], preferred_element_type=jnp.float32)
    aq_ref[0] = a
    gp_ref[0] = g + a + b1_ref[...]


def _precompute(pts, feats, w1a, w1b, b1r):
    B, P, _ = pts.shape
    C = w1b.shape[1]
    return pl.pallas_call(
        _pre_body,
        grid=(B,),
        in_specs=[
            pl.BlockSpec((1, P, 3), lambda b: (b, 0, 0)),
            pl.BlockSpec((1, P, C), lambda b: (b, 0, 0)),
            pl.BlockSpec((3, C), lambda b: (0, 0)),
            pl.BlockSpec((C, C), lambda b: (0, 0)),
            pl.BlockSpec((1, C), lambda b: (0, 0)),
        ],
        out_specs=[
            pl.BlockSpec((1, P, C), lambda b: (b, 0, 0)),
            pl.BlockSpec((1, P, C), lambda b: (b, 0, 0)),
        ],
        out_shape=[
            jax.ShapeDtypeStruct((B, P, C), jnp.float32),
            jax.ShapeDtypeStruct((B, P, C), jnp.float32),
        ],
    )(pts, feats, w1a, w1b, b1r)


def _topk_body(K, P, base, qoff, ptsq_ref, ptsT_ref, idx_ref):
    q = ptsq_ref[0]
    pT = ptsT_ref[0]
    BQ = q.shape[0]
    d = None
    for c in range(3):
        t = q[:, c:c + 1] - pT[c:c + 1, :]
        t = t * t
        d = t if d is None else d + t
    rowloc = (lax.broadcasted_iota(jnp.int32, (BQ, 1), 0)
              + pl.program_id(1) * BQ + qoff)
    idx_ref[0, :, 0:1] = rowloc + base
    r_i = lax.broadcasted_iota(jnp.int32, (P, 8), 0)
    r_j = lax.broadcasted_iota(jnp.int32, (P, 8), 1)
    iomat = jnp.where(r_j == 0, r_i >> 6,
                      jnp.where(r_j == 1, r_i & 63, 0)).astype(jnp.float32)
    inf = jnp.float32(float("inf"))
    iota = lax.broadcasted_iota(jnp.int32, (BQ, P), 1)
    d = jnp.where(iota == rowloc, inf, d)
    for k in range(1, K):
        m = jnp.min(d, axis=1, keepdims=True)
        eq = d == m
        imf = jnp.dot(eq.astype(jnp.float32), iomat,
                      preferred_element_type=jnp.float32)
        im = imf[:, :1].astype(jnp.int32) * 64 + imf[:, 1:2].astype(jnp.int32)
        idx_ref[0, :, k:k + 1] = im + base
        if k < K - 1:
            d = jnp.where(eq, inf, d)


def _topk(ptsq, ptsT, K, BQ, base=0, qoff=0):
    _, PQ, _ = ptsq.shape
    P = ptsT.shape[2]
    return pl.pallas_call(
        functools.partial(_topk_body, K, P, base, qoff),
        grid=(1, PQ // BQ),
        in_specs=[
            pl.BlockSpec((1, BQ, 3), lambda b, i: (b, i, 0)),
            pl.BlockSpec((1, 3, P), lambda b, i: (b, 0, 0)),
        ],
        out_specs=pl.BlockSpec((1, BQ, K), lambda b, i: (b, i, 0)),
        out_shape=jax.ShapeDtypeStruct((1, PQ, K), jnp.int32),
    )(ptsq, ptsT)


def _sc_gather(table, idxg, CH=128):
    N = idxg.shape[0]
    C = table.shape[1]
    info = plsc.get_sparse_core_info()
    NC, NS = info.num_cores, info.num_subcores
    NW = NC * NS
    n_per_w = N // NW
    nch = n_per_w // CH
    mesh = plsc.VectorSubcoreMesh(core_axis_name="c", subcore_axis_name="s")

    @functools.partial(
        pl.kernel,
        out_type=jax.ShapeDtypeStruct((N, C), jnp.float32),
        mesh=mesh,
        scratch_types=[
            pltpu.VMEM((CH,), jnp.int32),
            pltpu.VMEM((CH, C), jnp.float32),
            pltpu.SemaphoreType.DMA,
        ],
    )
    def k(table_hbm, idx_hbm, out_hbm, idx_v, rows_v, sem):
        wid = lax.axis_index("s") * NC + lax.axis_index("c")
        base = wid * n_per_w

        nrow = table_hbm.shape[0]

        def body(i, _):
            off = base + i * CH
            pltpu.sync_copy(idx_hbm.at[pl.ds(off, CH)], idx_v)
            for j in range(CH // 16):
                s = pl.ds(j * 16, 16)
                iv = idx_v[s]
                idx_v[s] = jnp.minimum(jnp.maximum(iv, 0), nrow - 1)
            pltpu.async_copy(table_hbm.at[idx_v], rows_v, sem).wait()
            pltpu.sync_copy(rows_v, out_hbm.at[pl.ds(off, CH)])
            return 0

        lax.fori_loop(0, nch, body, 0)

    return k(table, idxg)


def _mlp_body(K, C, nf_ref, aq_ref, w2_ref, b2_ref, out_ref):
    BQ = aq_ref.shape[0]
    nf = nf_ref[...].reshape(BQ, K, C)
    aq = aq_ref[...]
    h = jnp.maximum(nf - aq[:, None, :], 0.0)
    u = jnp.dot(h.reshape(BQ * K, C), w2_ref[...],
                preferred_element_type=jnp.float32) + b2_ref[...]
    out_ref[...] = jnp.max(u.reshape(BQ, K, C), axis=1)


def _mlp(NF, Aq2, W2, b2r, K, BQ):
    NT, C = Aq2.shape
    return pl.pallas_call(
        functools.partial(_mlp_body, K, C),
        grid=(NT // BQ,),
        in_specs=[
            pl.BlockSpec((BQ * K, C), lambda i: (i, 0)),
            pl.BlockSpec((BQ, C), lambda i: (i, 0)),
            pl.BlockSpec((C, C), lambda i: (0, 0)),
            pl.BlockSpec((1, C), lambda i: (0, 0)),
        ],
        out_specs=pl.BlockSpec((BQ, C), lambda i: (i, 0)),
        out_shape=jax.ShapeDtypeStruct((NT, C), jnp.float32),
    )(NF, Aq2, W2, b2r)


def kernel(pts, feats, W1, b1, W2, b2):
    B, P, _ = pts.shape
    C = feats.shape[-1]
    K = 16
    w1a = W1[:3]
    w1b = W1[3:]
    ptsT = jnp.transpose(pts, (0, 2, 1))
    Gp, Aq = _precompute(pts, feats, w1a, w1b, b1.reshape(1, C))
    Gp2 = Gp.reshape(B * P, C)
    Aq3 = Aq.reshape(B, P, C)
    NS = 2
    PS = P // NS
    outs = []
    for b in range(B):
        bouts = []
        for s in range(NS):
            idxs = _topk(pts[b:b + 1, s * PS:(s + 1) * PS], ptsT[b:b + 1],
                         K, BQ=256, base=b * P, qoff=s * PS)
            NFs = _sc_gather(Gp2, idxs.reshape(PS * K))
            bouts.append(_mlp(NFs, Aq3[b, s * PS:(s + 1) * PS], W2,
                              b2.reshape(1, C), K, BQ=128))
        outs.append(jnp.concatenate(bouts, axis=0))
    return jnp.stack(outs, axis=0)


# speedup vs baseline: 20.4156x; 1.0346x over previous
import functools

import jax
import jax.numpy as jnp
from jax import lax
from jax.experimental import pallas as pl
from jax.experimental.pallas import tpu as pltpu
from jax.experimental.pallas import tpu_sc as plsc


def _pre_body(pts_ref, feats_ref, w1a_ref, w1b_ref, b1_ref, gp_ref, aq_ref):
    p = pts_ref[0]
    f = feats_ref[0]
    a = jnp.dot(p, w1a_ref[...], preferred_element_type=jnp.float32)
    g = jnp.dot(f, w1b_ref[...], preferred_element_type=jnp.float32)
    aq_ref[0] = a
    gp = g + a + b1_ref[...]
    C = gp.shape[-1]
    def rne(x):
        b = lax.bitcast_convert_type(x, jnp.int32)
        return (b + 0x7FFF + ((b >> 16) & 1)) >> 16
    lo = rne(gp[:, :C // 2]) & jnp.int32(0xFFFF)
    hi = rne(gp[:, C // 2:])
    gp_ref[0] = lo | (hi << 16)


def _precompute(pts, feats, w1a, w1b, b1r):
    B, P, _ = pts.shape
    C = w1b.shape[1]
    return pl.pallas_call(
        _pre_body,
        grid=(B,),
        in_specs=[
            pl.BlockSpec((1, P, 3), lambda b: (b, 0, 0)),
            pl.BlockSpec((1, P, C), lambda b: (b, 0, 0)),
            pl.BlockSpec((3, C), lambda b: (0, 0)),
            pl.BlockSpec((C, C), lambda b: (0, 0)),
            pl.BlockSpec((1, C), lambda b: (0, 0)),
        ],
        out_specs=[
            pl.BlockSpec((1, P, C // 2), lambda b: (b, 0, 0)),
            pl.BlockSpec((1, P, C), lambda b: (b, 0, 0)),
        ],
        out_shape=[
            jax.ShapeDtypeStruct((B, P, C // 2), jnp.int32),
            jax.ShapeDtypeStruct((B, P, C), jnp.float32),
        ],
    )(pts, feats, w1a, w1b, b1r)


def _topk_body(K, P, base, qoff, ptsq_ref, ptsT_ref, idx_ref):
    q = ptsq_ref[0]
    pT = ptsT_ref[0]
    BQ = q.shape[0]
    d = None
    for c in range(3):
        t = q[:, c:c + 1] - pT[c:c + 1, :]
        t = t * t
        d = t if d is None else d + t
    rowloc = (lax.broadcasted_iota(jnp.int32, (BQ, 1), 0)
              + pl.program_id(1) * BQ + qoff)
    idx_ref[0, :, 0:1] = rowloc + base
    r_i = lax.broadcasted_iota(jnp.int32, (P, 8), 0)
    r_j = lax.broadcasted_iota(jnp.int32, (P, 8), 1)
    iomat = jnp.where(r_j == 0, r_i >> 6,
                      jnp.where(r_j == 1, r_i & 63, 0)).astype(jnp.float32)
    inf = jnp.float32(float("inf"))
    iota = lax.broadcasted_iota(jnp.int32, (BQ, P), 1)
    d = jnp.where(iota == rowloc, inf, d)
    for k in range(1, K):
        m = jnp.min(d, axis=1, keepdims=True)
        eq = d == m
        imf = jnp.dot(eq.astype(jnp.float32), iomat,
                      preferred_element_type=jnp.float32)
        im = imf[:, :1].astype(jnp.int32) * 64 + imf[:, 1:2].astype(jnp.int32)
        idx_ref[0, :, k:k + 1] = im + base
        if k < K - 1:
            d = jnp.where(eq, inf, d)


def _topk(ptsq, ptsT, K, BQ, base=0, qoff=0):
    _, PQ, _ = ptsq.shape
    P = ptsT.shape[2]
    return pl.pallas_call(
        functools.partial(_topk_body, K, P, base, qoff),
        grid=(1, PQ // BQ),
        in_specs=[
            pl.BlockSpec((1, BQ, 3), lambda b, i: (b, i, 0)),
            pl.BlockSpec((1, 3, P), lambda b, i: (b, 0, 0)),
        ],
        out_specs=pl.BlockSpec((1, BQ, K), lambda b, i: (b, i, 0)),
        out_shape=jax.ShapeDtypeStruct((1, PQ, K), jnp.int32),
    )(ptsq, ptsT)


def _sc_gather(table, idxg, CH=128):
    N = idxg.shape[0]
    C = table.shape[1]
    info = plsc.get_sparse_core_info()
    NC, NS = info.num_cores, info.num_subcores
    NW = NC * NS
    n_per_w = N // NW
    nch = n_per_w // CH
    mesh = plsc.VectorSubcoreMesh(core_axis_name="c", subcore_axis_name="s")

    dt = table.dtype

    @functools.partial(
        pl.kernel,
        out_type=jax.ShapeDtypeStruct((N, C), dt),
        mesh=mesh,
        scratch_types=[
            pltpu.VMEM((CH,), jnp.int32),
            pltpu.VMEM((CH, C), dt),
            pltpu.SemaphoreType.DMA,
        ],
    )
    def k(table_hbm, idx_hbm, out_hbm, idx_v, rows_v, sem):
        wid = lax.axis_index("s") * NC + lax.axis_index("c")
        base = wid * n_per_w

        nrow = table_hbm.shape[0]

        def body(i, _):
            off = base + i * CH
            pltpu.sync_copy(idx_hbm.at[pl.ds(off, CH)], idx_v)
            for j in range(CH // 16):
                s = pl.ds(j * 16, 16)
                iv = idx_v[s]
                idx_v[s] = jnp.minimum(jnp.maximum(iv, 0), nrow - 1)
            pltpu.async_copy(table_hbm.at[idx_v], rows_v, sem).wait()
            pltpu.sync_copy(rows_v, out_hbm.at[pl.ds(off, CH)])
            return 0

        lax.fori_loop(0, nch, body, 0)

    return k(table, idxg)


def _mlp_body(K, C, nf_ref, aq_ref, w2_ref, b2_ref, out_ref):
    BQ = aq_ref.shape[0]
    w = nf_ref[...]
    f0 = lax.bitcast_convert_type(w << 16, jnp.float32)
    f1 = lax.bitcast_convert_type(w & jnp.int32(-65536), jnp.float32)
    nf = jnp.concatenate([f0, f1], axis=-1).reshape(BQ, K, C)
    aq = aq_ref[...]
    h = jnp.maximum(nf - aq[:, None, :], 0.0)
    u = jnp.dot(h.reshape(BQ * K, C), w2_ref[...],
                preferred_element_type=jnp.float32) + b2_ref[...]
    out_ref[...] = jnp.max(u.reshape(BQ, K, C), axis=1)


def _mlp(NF, Aq2, W2, b2r, K, BQ):
    NT, C = Aq2.shape
    return pl.pallas_call(
        functools.partial(_mlp_body, K, C),
        grid=(NT // BQ,),
        in_specs=[
            pl.BlockSpec((BQ * K, C // 2), lambda i: (i, 0)),
            pl.BlockSpec((BQ, C), lambda i: (i, 0)),
            pl.BlockSpec((C, C), lambda i: (0, 0)),
            pl.BlockSpec((1, C), lambda i: (0, 0)),
        ],
        out_specs=pl.BlockSpec((BQ, C), lambda i: (i, 0)),
        out_shape=jax.ShapeDtypeStruct((NT, C), jnp.float32),
    )(NF, Aq2, W2, b2r)


def kernel(pts, feats, W1, b1, W2, b2):
    B, P, _ = pts.shape
    C = feats.shape[-1]
    K = 16
    w1a = W1[:3]
    w1b = W1[3:]
    ptsT = jnp.transpose(pts, (0, 2, 1))
    Gp, Aq = _precompute(pts, feats, w1a, w1b, b1.reshape(1, C))
    Gp2 = Gp.reshape(B * P, C // 2)
    Aq3 = Aq.reshape(B, P, C)
    NS = 2
    PS = P // NS
    outs = []
    for b in range(B):
        bouts = []
        for s in range(NS):
            idxs = _topk(pts[b:b + 1, s * PS:(s + 1) * PS], ptsT[b:b + 1],
                         K, BQ=256, base=b * P, qoff=s * PS)
            NFs = _sc_gather(Gp2, idxs.reshape(PS * K))
            bouts.append(_mlp(NFs, Aq3[b, s * PS:(s + 1) * PS], W2,
                              b2.reshape(1, C), K, BQ=128))
        outs.append(jnp.concatenate(bouts, axis=0))
    return jnp.stack(outs, axis=0)
